# Initial kernel scaffold; baseline (speedup 1.0000x reference)
import jax
import jax.numpy as jnp
from jax.experimental import pallas as pl


def kernel(inputs, targets):
    raise NotImplementedError("write your pallas kernel here")


# speedup vs baseline: 1.0730x; 1.0730x over previous
import functools

import jax
import jax.numpy as jnp
from jax.experimental import pallas as pl
from jax.experimental.pallas import tpu as pltpu

_ALPHA = 0.25
_OHEM_RATIO = 0.7


def _fused_body(tgt_ref, x_ref, out_ref, facc_ref, *, n_rows, n_cols,
                n_blocks, k):
    i = pl.program_id(0)
    x = x_ref[...]
    t = tgt_ref[0, 0, :]
    m = jnp.max(x, axis=1, keepdims=True)
    s = jnp.sum(jnp.exp(x - m), axis=1)
    lse = m[:, 0] + jnp.log(s)
    col = jax.lax.broadcasted_iota(jnp.int32, (n_rows, n_cols), 1)
    tl = jnp.sum(jnp.where(col == t[:, None], x, 0.0), axis=1)
    ce = lse - tl
    pt = jnp.exp(-ce)
    one_m = 1.0 - pt
    f = _ALPHA * one_m * one_m * ce
    facc_ref[pl.ds(i, 1), :] = f[None, :]

    @pl.when(i == n_blocks - 1)
    def _select():
        fall = facc_ref[...]
        bits = jax.lax.bitcast_convert_type(fall, jnp.int32)

        def step(j, prefix):
            cand = prefix | (jnp.int32(1) << (jnp.int32(30) - j))
            cnt = jnp.sum((bits >= cand).astype(jnp.int32))
            return jnp.where(cnt >= k, cand, prefix)

        thr = jax.lax.fori_loop(0, 31, step, jnp.int32(0))
        gt = bits > thr
        cnt_gt = jnp.sum(gt.astype(jnp.int32))
        sum_gt = jnp.sum(jnp.where(gt, fall, 0.0))
        thr_f = jnp.max(jnp.where(bits == thr, fall, 0.0))
        res = (
            sum_gt + (jnp.int32(k) - cnt_gt).astype(jnp.float32) * thr_f
        ) / jnp.float32(k)
        out_ref[...] = res[None, None]


def kernel(inputs, targets):
    n, c = inputs.shape
    r = 512
    g = n // r
    k = int(_OHEM_RATIO * n)
    tgt = targets.astype(jnp.int32).reshape(g, 1, r)
    body = functools.partial(_fused_body, n_rows=r, n_cols=c, n_blocks=g, k=k)
    out = pl.pallas_call(
        body,
        grid=(g,),
        in_specs=[
            pl.BlockSpec((1, 1, r), lambda i: (i, 0, 0)),
            pl.BlockSpec((r, c), lambda i: (i, 0)),
        ],
        out_specs=pl.BlockSpec((1, 1), lambda i: (0, 0)),
        out_shape=jax.ShapeDtypeStruct((1, 1), jnp.float32),
        scratch_shapes=[pltpu.VMEM((g, r), jnp.float32)],
    )(tgt, inputs)
    return out[0, 0]


# speedup vs baseline: 3.4715x; 3.2352x over previous
import functools

import jax
import jax.numpy as jnp
from jax.experimental import pallas as pl
from jax.experimental.pallas import tpu as pltpu

_ALPHA = 0.25
_OHEM_RATIO = 0.7


def _fused_body(tgt_ref, xt_ref, out_ref, facc_ref, *, n_classes, blk,
                n_blocks, k):
    i = pl.program_id(0)
    x = xt_ref[...]
    t = tgt_ref[0, 0, :]
    m = jnp.max(x, axis=0)
    s = jnp.sum(jnp.exp(x - m[None, :]), axis=0)
    lse = m + jnp.log(s)
    row = jax.lax.broadcasted_iota(jnp.int32, (n_classes, blk), 0)
    tl = jnp.sum(jnp.where(row == t[None, :], x, 0.0), axis=0)
    ce = lse - tl
    pt = jnp.exp(-ce)
    one_m = 1.0 - pt
    f = _ALPHA * one_m * one_m * ce
    facc_ref[pl.ds(i, 1), :] = f[None, :]

    @pl.when(i == n_blocks - 1)
    def _select():
        fall = facc_ref[...]
        bits = jax.lax.bitcast_convert_type(fall, jnp.int32)

        def step(j, prefix):
            cand = prefix | (jnp.int32(1) << (jnp.int32(30) - j))
            cnt = jnp.sum((bits >= cand).astype(jnp.int32))
            return jnp.where(cnt >= k, cand, prefix)

        thr = jax.lax.fori_loop(0, 31, step, jnp.int32(0))
        gt = bits > thr
        cnt_gt = jnp.sum(gt.astype(jnp.int32))
        sum_gt = jnp.sum(jnp.where(gt, fall, 0.0))
        thr_f = jnp.max(jnp.where(bits == thr, fall, 0.0))
        res = (
            sum_gt + (jnp.int32(k) - cnt_gt).astype(jnp.float32) * thr_f
        ) / jnp.float32(k)
        out_ref[...] = res[None, None]


def kernel(inputs, targets):
    n, c = inputs.shape
    blk = 2048
    g = n // blk
    k = int(_OHEM_RATIO * n)
    xt = inputs.T
    tgt = targets.astype(jnp.int32).reshape(g, 1, blk)
    body = functools.partial(_fused_body, n_classes=c, blk=blk, n_blocks=g,
                             k=k)
    out = pl.pallas_call(
        body,
        grid=(g,),
        in_specs=[
            pl.BlockSpec((1, 1, blk), lambda i: (i, 0, 0)),
            pl.BlockSpec((c, blk), lambda i: (0, i)),
        ],
        out_specs=pl.BlockSpec((1, 1), lambda i: (0, 0)),
        out_shape=jax.ShapeDtypeStruct((1, 1), jnp.float32),
        scratch_shapes=[pltpu.VMEM((g, blk), jnp.float32)],
    )(tgt, xt)
    return out[0, 0]
